# baseline (speedup 1.0000x reference)
import functools

import jax
import jax.numpy as jnp
from jax import lax
from jax.experimental import pallas as pl
from jax.experimental.pallas import tpu as pltpu
from jax.experimental.pallas import tpu_sc as plsc

NN = 10000
NE = 160000
D = 256
DE = 16
HALF = D // 2
QW = D // 4
NC, NS = 2, 16
RPT = 624

CH_L = 80
NCH_L = (NE // NS) // CH_L
CH_E = 40
NCH_E = (NE // (NC * NS)) // CH_E

_MESH = plsc.VectorSubcoreMesh(
    core_axis_name="c", subcore_axis_name="s", num_cores=NC, num_subcores=NS)


def _copy_tile_rows(src, dst, s, src_base=0, dst_base=0):
  start = s * RPT
  pltpu.sync_copy(src.at[pl.ds(src_base + start, RPT)],
                  dst.at[pl.ds(dst_base + start, RPT)])

  @pl.when(s == NS - 1)
  def _():
    tail = NS * RPT
    pltpu.sync_copy(src.at[pl.ds(src_base + tail, NN - tail)],
                    dst.at[pl.ds(dst_base + tail, NN - tail)])


def _mm_body(a_ref, b_ref, o_ref):
  o_ref[...] = jnp.dot(a_ref[...], b_ref[...],
                       preferred_element_type=jnp.float32)


def _mm_bias_body(a_ref, b_ref, bias_ref, o_ref):
  o_ref[...] = jnp.dot(a_ref[...], b_ref[...],
                       preferred_element_type=jnp.float32) + bias_ref[...]


def _hm_body(a_ref, w_ref, wx_ref, o_ref):
  t = jnp.dot(a_ref[...], w_ref[...], preferred_element_type=jnp.float32)
  for q in range(4):
    o_ref[q] = jnp.dot(t, wx_ref[:, q * QW:(q + 1) * QW],
                       preferred_element_type=jnp.float32)


def _hm_matmul(a, w, wx, block_m=1000):
  m = a.shape[0]
  gm = m // block_m
  out = pl.pallas_call(
      _hm_body,
      grid=(gm,),
      in_specs=[
          pl.BlockSpec((block_m, D), lambda i: (i, 0)),
          pl.BlockSpec((D, D), lambda i: (0, 0)),
          pl.BlockSpec((D, D), lambda i: (0, 0)),
      ],
      out_specs=pl.BlockSpec((4, block_m, QW), lambda i: (0, i, 0)),
      out_shape=jax.ShapeDtypeStruct((4, m, QW), jnp.float32),
  )(a, w, wx)
  return out.reshape(4 * m, QW)


def _eap_body(a_ref, b_ref, bias_ref, dre_ref, o_ref):
  t = jnp.dot(a_ref[...], b_ref[...],
              preferred_element_type=jnp.float32) + bias_ref[...]
  t = t * dre_ref[...]
  for q in range(4):
    o_ref[q] = t[:, q * QW:(q + 1) * QW]


def _eap_matmul(a, b, bias, dre, block_m=2000):
  m = a.shape[0]
  gm = m // block_m
  out = pl.pallas_call(
      _eap_body,
      grid=(gm,),
      in_specs=[
          pl.BlockSpec((block_m, DE), lambda i: (i, 0)),
          pl.BlockSpec((DE, D), lambda i: (0, 0)),
          pl.BlockSpec((1, D), lambda i: (0, 0)),
          pl.BlockSpec((block_m, 1), lambda i: (i, 0)),
      ],
      out_specs=pl.BlockSpec((4, block_m, QW), lambda i: (0, i, 0)),
      out_shape=jax.ShapeDtypeStruct((4, m, QW), jnp.float32),
  )(a, b, bias.reshape(1, D), dre)
  return out.reshape(4 * m, QW)


def _matmul(a, b, bias=None, block_m=1000):
  m, k = a.shape
  nout = b.shape[1]
  gm = m // block_m
  in_specs = [
      pl.BlockSpec((block_m, k), lambda i: (i, 0)),
      pl.BlockSpec((k, nout), lambda i: (0, 0)),
  ]
  args = [a, b]
  body = _mm_body
  if bias is not None:
    in_specs.append(pl.BlockSpec((1, nout), lambda i: (0, 0)))
    args.append(bias.reshape(1, nout))
    body = _mm_bias_body
  return pl.pallas_call(
      body,
      grid=(gm,),
      in_specs=in_specs,
      out_specs=pl.BlockSpec((block_m, nout), lambda i: (i, 0)),
      out_shape=jax.ShapeDtypeStruct((m, nout), jnp.float32),
  )(*args)


EPT_E = NE // (NC * NS)


@functools.partial(
    pl.kernel,
    out_type=jax.ShapeDtypeStruct((NE,), jnp.float32),
    mesh=_MESH,
    scratch_types=[
        pltpu.VMEM((NN,), jnp.float32),
        pltpu.VMEM((EPT_E + 16,), jnp.int32),
        pltpu.VMEM((EPT_E + 16,), jnp.float32),
    ],
    compiler_params=pltpu.CompilerParams(needs_layout_passes=False),
)
def _dre_kernel(row_hbm, dinv_hbm, out_hbm, dinv_v, ri_all, dre_v):
  c = lax.axis_index("c")
  s = lax.axis_index("s")
  wid = c * NS + s
  ebase = wid * EPT_E
  ri_all[pl.ds(EPT_E - 8, 16)] = jnp.zeros((16,), jnp.int32)
  pltpu.sync_copy(dinv_hbm, dinv_v)
  pltpu.sync_copy(row_hbm.at[pl.ds(ebase, EPT_E)], ri_all.at[pl.ds(0, EPT_E)])

  def body(i, carry):
    sl = pl.ds(i * 16, 16)
    dre_v[sl] = plsc.load_gather(dinv_v, [ri_all[sl]])
    return carry

  lax.fori_loop(0, (EPT_E + 8) // 16, body, 0)
  pltpu.sync_copy(dre_v.at[pl.ds(0, EPT_E)], out_hbm.at[pl.ds(ebase, EPT_E)])


@functools.partial(
    pl.kernel,
    out_type=jax.ShapeDtypeStruct((NC * NN, DE), jnp.float32),
    mesh=_MESH,
    scratch_types=[
        pltpu.VMEM((CH_E,), jnp.int32),
        pltpu.VMEM((CH_E, DE), jnp.float32),
        pltpu.VMEM_SHARED((NN, DE), jnp.float32),
    ],
    compiler_params=pltpu.CompilerParams(use_tc_tiling_on_sc=False),
)
def _deg_kernel(row_hbm, zeros_hbm, out_hbm, ri, ones_v, acc):
  c = lax.axis_index("c")
  s = lax.axis_index("s")
  wid = c * NS + s
  for e in range(CH_E):
    ones_v[e] = jnp.full((DE,), 1.0, jnp.float32)
  _copy_tile_rows(zeros_hbm, acc, s)
  plsc.subcore_barrier()

  def chunk(j, carry):
    base = wid * (NE // (NC * NS)) + j * CH_E
    pltpu.sync_copy(row_hbm.at[pl.ds(base, CH_E)], ri)
    pltpu.sync_copy(ones_v, acc.at[ri], add=True)
    return carry

  lax.fori_loop(0, NCH_E, chunk, 0)
  plsc.subcore_barrier()
  _copy_tile_rows(acc, out_hbm, s, dst_base=c * NN)


EPT_L = NE // NS


def _make_layer_kernel(qbase):
  @functools.partial(
      pl.kernel,
      out_type=jax.ShapeDtypeStruct((NC * NN, QW), jnp.float32),
      mesh=_MESH,
      scratch_types=[
          pltpu.VMEM((EPT_L,), jnp.int32),
          pltpu.VMEM((EPT_L,), jnp.int32),
          pltpu.VMEM((CH_L,), jnp.int32),
          pltpu.VMEM((CH_L,), jnp.int32),
          pltpu.VMEM((CH_L,), jnp.int32),
          pltpu.VMEM((CH_L,), jnp.int32),
          pltpu.VMEM((CH_L, QW), jnp.float32),
          pltpu.VMEM((CH_L, QW), jnp.float32),
          pltpu.VMEM((CH_L, QW), jnp.float32),
          pltpu.VMEM((CH_L, QW), jnp.float32),
          pltpu.VMEM((CH_L, QW), jnp.float32),
          pltpu.VMEM((CH_L, QW), jnp.float32),
          pltpu.VMEM_SHARED((NN, QW), jnp.float32),
          pltpu.SemaphoreType.DMA,
          pltpu.SemaphoreType.DMA,
          pltpu.SemaphoreType.DMA,
          pltpu.SemaphoreType.DMA,
      ],
      compiler_params=pltpu.CompilerParams(
          needs_layout_passes=False, use_tc_tiling_on_sc=False),
  )
  def _k(hm_hbm, eap_hbm, row_hbm, col_hbm, zeros_hbm, out_hbm,
         rows_all, cols_all, ri0, ri1, ci0, ci1, hr0, hr1, eb0, eb1,
         mv0, mv1, acc, gsem0, gsem1, esem0, esem1):
    c = lax.axis_index("c")
    s = lax.axis_index("s")
    ebase = s * EPT_L
    bufs = ((ri0, ci0, hr0, eb0, mv0, gsem0, esem0),
            (ri1, ci1, hr1, eb1, mv1, gsem1, esem1))

    _copy_tile_rows(zeros_hbm, acc, s)
    pltpu.sync_copy(row_hbm.at[pl.ds(ebase, EPT_L)], rows_all)
    pltpu.sync_copy(col_hbm.at[pl.ds(ebase, EPT_L)], cols_all)
    off = (qbase + c) * NN

    def addoff(i, carry):
      sl = pl.ds(i * 16, 16)
      rows_all[sl] = rows_all[sl] + off
      return carry

    lax.fori_loop(0, EPT_L // 16, addoff, 0)
    plsc.subcore_barrier()

    def issue(j, b):
      ri, ci, hr, eb, mv, gsem, esem = b
      base = j * CH_L
      for k in range(CH_L // 16):
        dsl = pl.ds(k * 16, 16)
        ssl = pl.ds(base + k * 16, 16)
        ri[dsl] = rows_all[ssl]
        ci[dsl] = cols_all[ssl]
      pltpu.async_copy(hm_hbm.at[ri], hr, gsem)
      pltpu.async_copy(
          eap_hbm.at[pl.ds((qbase + c) * NE + ebase + base, CH_L)], eb, esem)

    def wait_inputs(j, b):
      ri, ci, hr, eb, mv, gsem, esem = b
      base = j * CH_L
      pltpu.make_async_copy(hm_hbm.at[ri], hr, gsem).wait()
      pltpu.make_async_copy(
          eap_hbm.at[pl.ds((qbase + c) * NE + ebase + base, CH_L)], eb,
          esem).wait()

    def run(j, b, nb):
      ri, ci, hr, eb, mv, gsem, esem = b

      @pl.when(j + 1 < NCH_L)
      def _():
        issue(j + 1, nb)

      wait_inputs(j, b)

      def edge(e, inner):
        for l in range(QW // 16):
          sl = pl.ds(l * 16, 16)
          mv[e, sl] = jnp.maximum(hr[e, sl] + eb[e, sl], 0.0)
        return inner

      lax.fori_loop(0, CH_L, edge, 0)
      pltpu.sync_copy(mv, acc.at[ci], add=True)

    issue(0, bufs[0])

    def body(j, carry):
      @pl.when(j % 2 == 0)
      def _():
        run(j, bufs[0], bufs[1])

      @pl.when(j % 2 == 1)
      def _():
        run(j, bufs[1], bufs[0])

      return carry

    lax.fori_loop(0, NCH_L, body, 0)
    plsc.subcore_barrier()
    _copy_tile_rows(acc, out_hbm, s, dst_base=c * NN)

  return _k


_layer_q01 = _make_layer_kernel(0)
_layer_q23 = _make_layer_kernel(2)


@functools.partial(
    pl.kernel,
    out_type=jax.ShapeDtypeStruct((NE, DE), jnp.float32),
    mesh=_MESH,
    scratch_types=[
        pltpu.VMEM((CH_E,), jnp.int32),
        pltpu.VMEM((CH_E,), jnp.int32),
        pltpu.VMEM((CH_E, DE), jnp.float32),
        pltpu.VMEM((CH_E, DE), jnp.float32),
        pltpu.VMEM((CH_E, DE), jnp.float32),
        pltpu.VMEM((CH_E, DE), jnp.float32),
        pltpu.SemaphoreType.DMA,
    ],
    compiler_params=pltpu.CompilerParams(use_tc_tiling_on_sc=False),
)
def _edge_update_kernel(g_hbm, eaw_hbm, row_hbm, col_hbm, out_hbm,
                        ri, ci, gr, gc, ew, ov, sem):
  c = lax.axis_index("c")
  s = lax.axis_index("s")
  wid = c * NS + s

  def chunk(j, carry):
    base = wid * (NE // (NC * NS)) + j * CH_E
    pltpu.sync_copy(row_hbm.at[pl.ds(base, CH_E)], ri)
    pltpu.sync_copy(col_hbm.at[pl.ds(base, CH_E)], ci)
    pltpu.async_copy(g_hbm.at[ri], gr, sem).wait()
    pltpu.async_copy(g_hbm.at[ci], gc, sem).wait()
    pltpu.sync_copy(eaw_hbm.at[pl.ds(base, CH_E)], ew)
    for e in range(CH_E):
      ov[e] = jnp.maximum(gr[e] + gc[e] + ew[e], 0.0)
    pltpu.sync_copy(ov, out_hbm.at[pl.ds(base, CH_E)])
    return carry

  lax.fori_loop(0, NCH_E, chunk, 0)


def kernel(x, edge_attr, edge_index, W0, b0, Wm0, bm0, W1, b1, Wm1, bm1,
           Weu0, beu0, Weu1, beu1):
  row = edge_index[0]
  col = edge_index[1]
  zeros_l = jnp.zeros((NN, QW), jnp.float32)
  zeros_e = jnp.zeros((NN, DE), jnp.float32)

  degout = _deg_kernel(row, zeros_e)
  deg = degout[:NN, 0] + degout[NN:, 0]
  dinv = jnp.where(deg > 0, lax.rsqrt(jnp.maximum(deg, 1e-12)), 0.0)
  dre = _dre_kernel(row, dinv).reshape(NE, 1)
  dcol = dinv[:, None]

  def conv(xin, ea, W, b, Wm, bm):
    hm = _hm_matmul(xin * dcol, W, Wm[:D])
    eap = _eap_matmul(ea, Wm[D:], bm, dre)
    a01 = _layer_q01(hm, eap, row, col, zeros_l)
    a23 = _layer_q23(hm, eap, row, col, zeros_l)
    out = jnp.concatenate([a01[:NN], a01[NN:], a23[:NN], a23[NN:]], axis=1)
    return out * dcol + b

  def edge_update(xin, ea, Weu, beu):
    g = _matmul(xin, 0.5 * Weu[:D], block_m=1000)
    eaw = _matmul(ea, Weu[D:], bias=beu, block_m=2000)
    return _edge_update_kernel(g, eaw, row, col)

  x1 = conv(x, edge_attr, W0, b0, Wm0, bm0)
  ea1 = edge_update(x1, edge_attr, Weu0, beu0)
  x2 = conv(x1, ea1, W1, b1, Wm1, bm1)
  ea2 = edge_update(x2, ea1, Weu1, beu1)
  return (x2, ea2)
